# Initial kernel scaffold; baseline (speedup 1.0000x reference)
import jax
import jax.numpy as jnp
from jax.experimental import pallas as pl


def kernel(seq_embeds, table):
    raise NotImplementedError("write your pallas kernel here")


# speedup vs baseline: 1.3028x; 1.3028x over previous
import functools

import jax
import jax.numpy as jnp
from jax import lax
from jax.experimental import pallas as pl
from jax.experimental.pallas import tpu as pltpu
from jax.experimental.pallas import tpu_sc as plsc


@functools.cache
def _pos_embed_copy(num_rows, dim, dtype):
    info = plsc.get_sparse_core_info()
    nw = info.num_cores * info.num_subcores
    assert num_rows % nw == 0, (num_rows, nw)
    rows_per_w = num_rows // nw
    mesh = plsc.VectorSubcoreMesh(core_axis_name="c", subcore_axis_name="s")

    @functools.partial(
        pl.kernel,
        mesh=mesh,
        out_type=jax.ShapeDtypeStruct((num_rows, dim), dtype),
        scratch_types=[pltpu.VMEM((rows_per_w, dim), dtype)],
    )
    def k(table_hbm, out_hbm, buf):
        wid = lax.axis_index("s") * info.num_cores + lax.axis_index("c")
        base = wid * rows_per_w
        pltpu.sync_copy(table_hbm.at[pl.ds(base, rows_per_w)], buf)
        pltpu.sync_copy(buf, out_hbm.at[pl.ds(base, rows_per_w)])

    return k


def kernel(seq_embeds, table):
    len_seq = seq_embeds.shape[-2]
    pos_embeds = _pos_embed_copy(len_seq, table.shape[-1], table.dtype)(table)
    if seq_embeds.ndim == 3:
        pos_embeds = pos_embeds[None]
    return pos_embeds
